# baseline (speedup 1.0000x reference)
import jax
import jax.numpy as jnp
from jax.experimental import pallas as pl
from jax.experimental.pallas import tpu as pltpu

_ROWS = 128
_VOCAB = 100000
_CHUNK_ROWS = 8
_NCHUNKS = _ROWS // _CHUNK_ROWS
_NBUF = 4


def _body(x_hbm, o_hbm, *scratch):
    xbufs = scratch[:_NBUF]
    obufs = scratch[_NBUF:2 * _NBUF]
    ld_sem, st_sem = scratch[2 * _NBUF], scratch[2 * _NBUF + 1]

    def load(chunk, slot):
        return pltpu.make_async_copy(
            x_hbm.at[pl.ds(chunk * _CHUNK_ROWS, _CHUNK_ROWS), :],
            xbufs[slot],
            ld_sem.at[slot],
        )

    def store(chunk, slot):
        return pltpu.make_async_copy(
            obufs[slot],
            o_hbm.at[pl.ds(chunk * _CHUNK_ROWS, _CHUNK_ROWS), :],
            st_sem.at[slot],
        )

    for slot in range(_NBUF):
        load(slot, slot).start()

    for i in range(_NCHUNKS):
        slot = i % _NBUF
        load(i, slot).wait()
        if i >= _NBUF:
            store(i - _NBUF, slot).wait()
        x = xbufs[slot][...]
        m = jnp.max(x, axis=-1, keepdims=True)
        s = jnp.sum(jnp.exp(x - m), axis=-1, keepdims=True)
        obufs[slot][...] = (x - m) - jnp.log(s)
        store(i, slot).start()
        nxt = i + _NBUF
        if nxt < _NCHUNKS:
            load(nxt, slot).start()

    for i in range(_NCHUNKS - _NBUF, _NCHUNKS):
        store(i, i % _NBUF).wait()


def kernel(logits):
    scratch = (
        [pltpu.MemorySpace.VMEM((_CHUNK_ROWS, _VOCAB), jnp.float32)
         for _ in range(2 * _NBUF)]
        + [pltpu.SemaphoreType.DMA((_NBUF,)),
           pltpu.SemaphoreType.DMA((_NBUF,))]
    )
    return pl.pallas_call(
        _body,
        in_specs=[pl.BlockSpec(memory_space=pl.ANY)],
        out_specs=pl.BlockSpec(memory_space=pl.ANY),
        out_shape=jax.ShapeDtypeStruct((_ROWS, _VOCAB), logits.dtype),
        scratch_shapes=scratch,
    )(logits)
